# Initial kernel scaffold; baseline (speedup 1.0000x reference)
import jax
import jax.numpy as jnp
from jax.experimental import pallas as pl


def kernel(x, edge_index, batch, W1, b1, g1, be1, W2, b2, g_out, be_out, lin1_W, lin1_b, cls_W, cls_b):
    raise NotImplementedError("write your pallas kernel here")


# speedup vs baseline: 5.6422x; 5.6422x over previous
import functools

import jax
import jax.numpy as jnp
from jax import lax
from jax.experimental import pallas as pl
from jax.experimental.pallas import tpu as pltpu
from jax.experimental.pallas import tpu_sc as plsc

_BN_INV = 0.9999950000374997


def _make_sc_aggregate(N, D, E):
    NC, NS = 2, 16
    NT = NC * NS
    assert E % NT == 0
    ept = E // NT
    K = 128
    full = ept // K
    tail = ept % K
    rpt = (N // NS) & ~7
    rtail = N - rpt * NS

    mesh = plsc.VectorSubcoreMesh(core_axis_name="c", subcore_axis_name="s")

    scratch = [
        pltpu.VMEM((K,), jnp.int32),
        pltpu.VMEM((K,), jnp.int32),
        pltpu.VMEM((K, D), jnp.float32),
        pltpu.VMEM_SHARED((N, D), jnp.float32),
        pltpu.SemaphoreType.DMA,
    ]
    if tail:
        scratch += [
            pltpu.VMEM((tail,), jnp.int32),
            pltpu.VMEM((tail,), jnp.int32),
            pltpu.VMEM((tail, D), jnp.float32),
        ]

    @functools.partial(
        pl.kernel,
        out_type=jax.ShapeDtypeStruct((NC, N, D), jnp.float32),
        mesh=mesh,
        scratch_types=scratch,
    )
    def sc_agg(h_hbm, src_hbm, dst_hbm, zsrc_hbm, out_hbm,
               src_v, dst_v, rows_v, agg_sh, gsem, *tail_refs):
        c = lax.axis_index("c")
        s = lax.axis_index("s")
        tile = c * NS + s
        base = tile * ept

        rbase = s * rpt
        pltpu.sync_copy(zsrc_hbm.at[pl.ds(rbase, rpt)],
                        agg_sh.at[pl.ds(rbase, rpt)])
        if rtail:
            @pl.when(s == 0)
            def _():
                pltpu.sync_copy(zsrc_hbm.at[pl.ds(NS * rpt, rtail)],
                                agg_sh.at[pl.ds(NS * rpt, rtail)])
        plsc.subcore_barrier()

        def chunk(j, carry):
            off = base + j * K
            pltpu.sync_copy(src_hbm.at[pl.ds(off, K)], src_v)
            pltpu.sync_copy(dst_hbm.at[pl.ds(off, K)], dst_v)
            pltpu.async_copy(h_hbm.at[src_v], rows_v, gsem).wait()
            pltpu.sync_copy(rows_v, agg_sh.at[dst_v], add=True)
            return carry

        lax.fori_loop(0, full, chunk, 0)

        if tail:
            src_t, dst_t, rows_t = tail_refs
            off = base + full * K
            pltpu.sync_copy(src_hbm.at[pl.ds(off, tail)], src_t)
            pltpu.sync_copy(dst_hbm.at[pl.ds(off, tail)], dst_t)
            pltpu.async_copy(h_hbm.at[src_t], rows_t, gsem).wait()
            pltpu.sync_copy(rows_t, agg_sh.at[dst_t], add=True)

        plsc.subcore_barrier()
        pltpu.sync_copy(agg_sh.at[pl.ds(rbase, rpt)],
                        out_hbm.at[c, pl.ds(rbase, rpt)])
        if rtail:
            @pl.when(s == 0)
            def _():
                pltpu.sync_copy(agg_sh.at[pl.ds(NS * rpt, rtail)],
                                out_hbm.at[c, pl.ds(NS * rpt, rtail)])

    return sc_agg


def _tc_mlp(agg2, h, W1, b1, g1, be1, W2, b2, g2, be2, R=1000):
    N, D = h.shape

    def body(agg_ref, h_ref, W1_ref, b1_ref, s1_ref, be1_ref,
             W2_ref, b2_ref, s2_ref, be2_ref, out_ref):
        m = agg_ref[0] + agg_ref[1] + h_ref[...]
        t = jnp.dot(m, W1_ref[...], preferred_element_type=jnp.float32)
        t = (t + b1_ref[...]) * s1_ref[...] + be1_ref[...]
        t = jnp.maximum(t, 0.0)
        u = jnp.dot(t, W2_ref[...], preferred_element_type=jnp.float32)
        u = (u + b2_ref[...]) * s2_ref[...] + be2_ref[...]
        out_ref[...] = jnp.maximum(u, 0.0)

    s1 = (g1 * _BN_INV).reshape(1, D)
    s2 = (g2 * _BN_INV).reshape(1, D)
    grid = N // R
    vec = pl.BlockSpec((1, D), lambda i: (0, 0))
    mat = pl.BlockSpec((D, D), lambda i: (0, 0))
    return pl.pallas_call(
        body,
        grid=(grid,),
        in_specs=[
            pl.BlockSpec((2, R, D), lambda i: (0, i, 0)),
            pl.BlockSpec((R, D), lambda i: (i, 0)),
            mat, vec, vec, vec, mat, vec, vec, vec,
        ],
        out_specs=pl.BlockSpec((R, D), lambda i: (i, 0)),
        out_shape=jax.ShapeDtypeStruct((N, D), jnp.float32),
    )(agg2, h, W1, b1.reshape(1, D), s1, be1.reshape(1, D),
      W2, b2.reshape(1, D), s2, be2.reshape(1, D))


def _tc_pool_head(h1, h2, h3, batchf, G, lin1_W, lin1_b, cls_Wp, cls_bp,
                  R=1000):
    N, D = h1.shape

    def body(h1_ref, h2_ref, h3_ref, b_ref, lw_ref, lb_ref, cw_ref, cb_ref,
             out_ref, acc_ref):
        i = pl.program_id(0)

        @pl.when(i == 0)
        def _():
            acc_ref[...] = jnp.zeros_like(acc_ref)

        hs = h1_ref[...] + h2_ref[...] + h3_ref[...]
        gid = lax.broadcasted_iota(jnp.int32, (G, R), 0).astype(jnp.float32)
        P = jnp.where(gid == b_ref[0], 1.0, 0.0)
        acc_ref[...] += jnp.dot(P, hs, preferred_element_type=jnp.float32)

        @pl.when(i == pl.num_programs(0) - 1)
        def _():
            hh = jnp.dot(acc_ref[...], lw_ref[...],
                         preferred_element_type=jnp.float32) + lb_ref[...]
            hh = jnp.maximum(hh, 0.0)
            out_ref[...] = jnp.dot(hh, cw_ref[...],
                                   preferred_element_type=jnp.float32) + cb_ref[...]

    rows = pl.BlockSpec((R, D), lambda i: (i, 0))
    return pl.pallas_call(
        body,
        grid=(N // R,),
        in_specs=[
            rows, rows, rows,
            pl.BlockSpec((1, 1, R), lambda i: (i, 0, 0)),
            pl.BlockSpec((D, D), lambda i: (0, 0)),
            pl.BlockSpec((1, D), lambda i: (0, 0)),
            pl.BlockSpec((D, 128), lambda i: (0, 0)),
            pl.BlockSpec((1, 128), lambda i: (0, 0)),
        ],
        out_specs=pl.BlockSpec((G, 128), lambda i: (0, 0)),
        out_shape=jax.ShapeDtypeStruct((G, 128), jnp.float32),
        scratch_shapes=[pltpu.VMEM((G, D), jnp.float32)],
    )(h1, h2, h3, batchf, lin1_W, lin1_b.reshape(1, D), cls_Wp, cls_bp)


def kernel(x, edge_index, batch, W1, b1, g1, be1, W2, b2, g_out, be_out,
           lin1_W, lin1_b, cls_W, cls_b):
    N, D = x.shape
    E = edge_index.shape[1]
    L = W1.shape[0]
    G = 128

    src = edge_index[0]
    dst = edge_index[1]
    zsrc = jnp.zeros((N, D), jnp.float32)
    sc_agg = _make_sc_aggregate(N, D, E)

    hs = []
    h = x
    for i in range(L):
        agg2 = sc_agg(h, src, dst, zsrc)
        h = _tc_mlp(agg2, h, W1[i], b1[i], g1[i], be1[i],
                    W2[i], b2[i], g_out[i], be_out[i])
        hs.append(h)

    batchf = batch.astype(jnp.float32).reshape(N // 1000, 1, 1000)
    cls_Wp = jnp.pad(cls_W, ((0, 0), (0, 127)))
    cls_bp = jnp.pad(cls_b, (0, 127)).reshape(1, 128)
    out2d = _tc_pool_head(hs[0], hs[1], hs[2], batchf, G,
                          lin1_W, lin1_b, cls_Wp, cls_bp)
    return out2d[:, 0]
